# Initial kernel scaffold; baseline (speedup 1.0000x reference)
import jax
import jax.numpy as jnp
from jax.experimental import pallas as pl


def kernel(prob_map, rain_logit, pred_phys, label_map, rain_max_true, rain_spatial_true, phys_targets, phys_mu, phys_std):
    raise NotImplementedError("write your pallas kernel here")


# speedup vs baseline: 61.2645x; 61.2645x over previous
import jax
import jax.numpy as jnp
from jax import lax
from jax.experimental import pallas as pl
from jax.experimental.pallas import tpu as pltpu

_PW = 2.0
_ALPHA = 0.75
_HNM = 10
_TVA = 0.3
_TVB = 0.7


def _body(prob_ref, label_ref, rlog_ref, rsp_ref, pp_ref, pt_ref, mu_ref,
          std_ref, out_ref, scr_ref):
    b = pl.program_id(0)
    praw = prob_ref[0]
    t = label_ref[0]
    H, W = praw.shape
    N = H * W

    p = jnp.clip(praw, 1e-6, 1 - 1e-6)
    is_pos = t == 1.0
    p_t = jnp.where(is_pos, p, 1.0 - p)
    q = 1.0 - p_t
    coef = jnp.where(is_pos, _ALPHA * _PW, 1.0 - _ALPHA)
    focal = -(coef * q * q) * jnp.log(p_t)

    n_pos_f = jnp.sum(t)
    n_pos_i = jnp.sum(t.astype(jnp.int32))
    n_neg_i = N - n_pos_i
    n_hard_i = jnp.minimum(n_pos_i * _HNM, n_neg_i)
    sum_pos = jnp.sum(focal * t)
    neg_all = jnp.sum(focal * (1.0 - t))

    scr_ref[0] = neg_all

    @pl.when(n_hard_i < n_neg_i)
    def _topk_fallback():
        vals = jnp.where(is_pos, -1.0, focal)
        vbits = lax.bitcast_convert_type(vals, jnp.int32)
        k = n_hard_i

        def step(_, lh):
            lo, hi = lh
            mid = lo + (hi - lo + 1) // 2
            cnt = jnp.sum((vbits >= mid).astype(jnp.int32))
            take = cnt >= k
            return (jnp.where(take, mid, lo), jnp.where(take, hi, mid - 1))

        lo, _ = lax.fori_loop(0, 31, step, (jnp.int32(0), jnp.int32(0x7F7FFFFF)))
        gt = vbits > lo
        eq = vbits == lo
        cnt_gt = jnp.sum(gt.astype(jnp.int32))
        cnt_eq = jnp.maximum(jnp.sum(eq.astype(jnp.int32)), 1)
        sum_gt = jnp.sum(jnp.where(gt, focal, 0.0))
        sum_eq = jnp.sum(jnp.where(eq, focal, 0.0))
        scr_ref[0] = sum_gt + (k - cnt_gt).astype(jnp.float32) * sum_eq / cnt_eq.astype(jnp.float32)

    sum_hard = scr_ref[0]
    fl_b = (sum_pos + sum_hard) / (n_pos_f + n_hard_i.astype(jnp.float32))

    tp = jnp.sum(p * t)
    fp = jnp.sum(p) - tp
    fn = n_pos_f - tp
    tv_b = 1.0 - (tp + 1.0) / (tp + _TVA * fp + _TVB * fn + 1.0)

    r = rsp_ref[0]
    rlt = jnp.log1p(jnp.maximum(r, 0.0))
    gate = jnp.logical_or(praw > 0.1, r > 1.0).astype(jnp.float32)
    heavy = (r >= 50.0).astype(jnp.float32)
    w = gate * (1.0 + 3.0 * heavy)
    d = rlog_ref[0] - rlt
    ad = jnp.abs(d)
    hub = jnp.where(ad < 1.0, 0.5 * d * d, ad - 0.5)

    out_ref[0, 0, 0] = fl_b
    out_ref[0, 0, 1] = tv_b
    out_ref[0, 0, 2] = jnp.sum(hub * w)
    out_ref[0, 0, 3] = jnp.sum(w)

    @pl.when(b == 0)
    def _aux():
        norm = (pt_ref[...] - mu_ref[...]) / (std_ref[...] + 1e-6)
        norm = jnp.where(jnp.isnan(norm), 0.0, norm)
        out_ref[0, 0, 4] = jnp.mean((pp_ref[...] - norm) ** 2)

    @pl.when(b != 0)
    def _aux0():
        out_ref[0, 0, 4] = 0.0


def kernel(prob_map, rain_logit, pred_phys, label_map, rain_max_true,
           rain_spatial_true, phys_targets, phys_mu, phys_std):
    B, H, W = prob_map.shape
    P = pred_phys.shape[1]
    mu_b = jnp.broadcast_to(phys_mu[None, :], (B, P))
    std_b = jnp.broadcast_to(phys_std[None, :], (B, P))

    img = pl.BlockSpec((1, H, W), lambda b: (b, 0, 0))
    small = pl.BlockSpec((B, P), lambda b: (0, 0))
    stats = pl.pallas_call(
        _body,
        grid=(B,),
        in_specs=[img, img, img, img, small, small, small, small],
        out_specs=pl.BlockSpec((1, 1, 8), lambda b: (b, 0, 0),
                               memory_space=pltpu.SMEM),
        out_shape=jax.ShapeDtypeStruct((B, 1, 8), jnp.float32),
        scratch_shapes=[pltpu.SMEM((1,), jnp.float32)],
    )(prob_map, label_map, rain_logit, rain_spatial_true,
      pred_phys, phys_targets, mu_b, std_b)

    stats = stats[:, 0, :]
    fl = jnp.mean(stats[:, 0])
    tv = jnp.mean(stats[:, 1])
    reg = jnp.sum(stats[:, 2]) / jnp.maximum(jnp.sum(stats[:, 3]), 1.0)
    aux = stats[0, 4]
    total = fl + 0.5 * tv + 1.0 * reg + 0.1 * aux
    return (total, fl, tv, reg, aux)
